# baseline (speedup 1.0000x reference)
import functools
import math

import jax
import jax.numpy as jnp
from jax import lax
from jax.experimental import pallas as pl
from jax.experimental.pallas import tpu as pltpu
from jax.experimental.pallas import tpu_sc as plsc

B = 4
T = 1024
L = 6
D = 256
H = 8
DH = 32
DFF = 1024
LEVEL = 5120
N = B * LEVEL
BT = B * T

NC = 2
NS = 16
NW = NC * NS
CH = 128

_SC_MESH = dict(core_axis_name="c", subcore_axis_name="s", num_cores=NC,
                num_subcores=NS)


def _prep_body(row_ref, col_ref, g_ref, s_ref):
    p = pl.program_id(0)
    off = (p % B) * LEVEL
    trow = row_ref[0]
    tcol = col_ref[...]
    eq = tcol == trow
    it = lax.broadcasted_iota(jnp.int32, (T, T), 0)
    jt = lax.broadcasted_iota(jnp.int32, (T, T), 1)
    loser = jnp.any(eq & (jt > it), axis=1, keepdims=True)
    g = tcol + off
    g_ref[...] = g
    s_ref[...] = jnp.where(loser, N, g)


_index_prep = pl.pallas_call(
    _prep_body,
    grid=(L * B,),
    in_specs=[
        pl.BlockSpec((1, 1, T), lambda p: (p, 0, 0)),
        pl.BlockSpec((T, 1), lambda p: (p, 0)),
    ],
    out_specs=[
        pl.BlockSpec((T, 1), lambda p: (p, 0)),
        pl.BlockSpec((T, 1), lambda p: (p, 0)),
    ],
    out_shape=[
        jax.ShapeDtypeStruct((L * BT, 1), jnp.int32),
        jax.ShapeDtypeStruct((L * BT, 1), jnp.int32),
    ],
)


def _make_sc_gather(M):
    rw = M // NW
    ch = min(CH, rw)
    nch = rw // ch

    @functools.partial(
        pl.kernel,
        out_type=jax.ShapeDtypeStruct((M, D), jnp.float32),
        mesh=plsc.VectorSubcoreMesh(**_SC_MESH),
        scratch_types=[
            pltpu.VMEM((ch,), jnp.int32),
            pltpu.VMEM((ch, D), jnp.float32),
            pltpu.SemaphoreType.DMA,
        ],
    )
    def gk(table, idx, out, idx_v, rows_v, sem):
        wid = lax.axis_index("s") * NC + lax.axis_index("c")
        for c in range(nch):
            base = wid * rw + c * ch
            pltpu.sync_copy(idx.at[pl.ds(base, ch)], idx_v)
            pltpu.async_copy(table.at[idx_v], rows_v, sem).wait()
            pltpu.sync_copy(rows_v, out.at[pl.ds(base, ch)])

    return gk


_gather_half = _make_sc_gather(2 * T)
_gather_pe = _make_sc_gather(L * BT)

RW = 2 * T // NW


@functools.partial(
    pl.kernel,
    out_type=(),
    mesh=plsc.VectorSubcoreMesh(**_SC_MESH),
    scratch_types=[
        pltpu.VMEM((RW,), jnp.int32),
        pltpu.VMEM((RW, D), jnp.float32),
        pltpu.SemaphoreType.DMA,
    ],
)
def _sc_scatter(x, sidx, table, idx_v, rows_v, sem):
    wid = lax.axis_index("s") * NC + lax.axis_index("c")
    base = wid * RW
    pltpu.sync_copy(sidx.at[pl.ds(base, RW)], idx_v)
    pltpu.sync_copy(x.at[pl.ds(base, RW)], rows_v)
    pltpu.async_copy(rows_v, table.at[idx_v], sem).wait()


@functools.partial(
    pl.kernel,
    out_type=jax.ShapeDtypeStruct((2 * T, D), jnp.float32),
    mesh=plsc.VectorSubcoreMesh(**_SC_MESH),
    scratch_types=[
        pltpu.VMEM((RW,), jnp.int32),
        pltpu.VMEM((RW,), jnp.int32),
        pltpu.VMEM((RW, D), jnp.float32),
        pltpu.VMEM((RW, D), jnp.float32),
        pltpu.SemaphoreType.DMA,
    ],
)
def _sc_scatter_gather(x, sidx, gidx, table, qout, sidx_v, gidx_v, srows_v,
                       grows_v, sem):
    c = lax.axis_index("c")
    s = lax.axis_index("s")
    base = c * T + s * RW
    pltpu.sync_copy(sidx.at[pl.ds(base, RW)], sidx_v)
    pltpu.sync_copy(gidx.at[pl.ds(base, RW)], gidx_v)
    pltpu.sync_copy(x.at[pl.ds(base, RW)], srows_v)
    pltpu.async_copy(srows_v, table.at[sidx_v], sem).wait()
    plsc.subcore_barrier()
    pltpu.async_copy(table.at[gidx_v], grows_v, sem).wait()
    pltpu.sync_copy(grows_v, qout.at[pl.ds(base, RW)])


def _ln(x):
    m = jnp.mean(x, axis=1, keepdims=True)
    c = x - m
    v = jnp.mean(c * c, axis=1, keepdims=True)
    return c * lax.rsqrt(v + 1e-5)


def _layer_body(q_ref, pe_ref, xy_ref, sc_ref, wq_ref, wk_ref, wv_ref,
                wo_ref, w1_ref, w2_ref, wxy_ref, wp_ref, o_ref):
    f32 = jnp.float32
    bf = jnp.bfloat16
    q = q_ref[...]
    pe = pe_ref[...] + jnp.dot(xy_ref[...], wxy_ref[...],
                               preferred_element_type=f32)
    qk = (q + pe).astype(bf)
    wq = wq_ref[0].astype(bf)
    wk = wk_ref[0].astype(bf)
    wv = wv_ref[0].astype(bf)
    wo = wo_ref[0].astype(bf)
    Q = jnp.dot(qk, wq, preferred_element_type=f32)
    K = jnp.dot(qk, wk, preferred_element_type=f32).astype(bf)
    V = jnp.dot(q.astype(bf), wv, preferred_element_type=f32).astype(bf)
    logit = jnp.sum(q * wp_ref[0:1, :], axis=1, keepdims=True)
    gate = sc_ref[...] * jax.nn.sigmoid(logit)
    scale = 1.0 / math.sqrt(DH)
    outs = []
    for h in range(H):
        sl = slice(h * DH, (h + 1) * DH)
        S = lax.dot_general((Q[:, sl] * scale).astype(bf), K[:, sl],
                            (((1,), (1,)), ((), ())),
                            preferred_element_type=f32)
        E = jnp.exp(S)
        P = (E / jnp.sum(E, axis=1, keepdims=True)).astype(bf)
        outs.append(jnp.dot(P, V[:, sl], preferred_element_type=f32))
    attn = jnp.concatenate(outs, axis=1).astype(bf)
    out = jnp.dot(attn, wo, preferred_element_type=f32)
    x = _ln(q + out * gate)
    ff = jnp.dot(jax.nn.relu(jnp.dot(x.astype(bf), w1_ref[0].astype(bf),
                                     preferred_element_type=f32)).astype(bf),
                 w2_ref[0].astype(bf), preferred_element_type=f32)
    o_ref[...] = _ln(x + ff)


def _make_layer(l, h):
    return pl.pallas_call(
        _layer_body,
        grid=(2,),
        in_specs=[
            pl.BlockSpec((T, D), lambda b: (b, 0)),
            pl.BlockSpec((T, D), lambda b: (l * B + 2 * h + b, 0)),
            pl.BlockSpec((T, 8), lambda b: (l * B + 2 * h + b, 0)),
            pl.BlockSpec((T, 1), lambda b: (l * B + 2 * h + b, 0)),
            pl.BlockSpec((1, D, D), lambda b: (l, 0, 0)),
            pl.BlockSpec((1, D, D), lambda b: (l, 0, 0)),
            pl.BlockSpec((1, D, D), lambda b: (l, 0, 0)),
            pl.BlockSpec((1, D, D), lambda b: (l, 0, 0)),
            pl.BlockSpec((1, D, DFF), lambda b: (l, 0, 0)),
            pl.BlockSpec((1, DFF, D), lambda b: (l, 0, 0)),
            pl.BlockSpec((8, D), lambda b: (0, 0)),
            pl.BlockSpec((8, D), lambda b: (0, 0)),
        ],
        out_specs=pl.BlockSpec((T, D), lambda b: (b, 0)),
        out_shape=jax.ShapeDtypeStruct((2 * T, D), jnp.float32),
    )


_layers = [[_make_layer(l, h) for h in (0, 1)] for l in range(L)]


def kernel(feature_values, pos_values, token_indices, token_scores,
           xy_positions, Wq, Wk, Wv, Wo, W1, b1, W2, b2, g1, be1, g2, be2,
           Wxy, Wp, bp):
    f32 = jnp.float32
    tok = token_indices.astype(jnp.int32)
    gidx_col, sidx_col = _index_prep(tok.reshape(L * B, 1, T),
                                     tok.reshape(L * BT, 1))
    gidx = gidx_col.reshape(L, BT)
    sidx = sidx_col.reshape(L, BT)

    xy_pad = jnp.pad(xy_positions, ((0, 0), (0, 0), (0, 0), (0, 6)))
    xy_pad = xy_pad.reshape(L * BT, 8).astype(f32)
    wxy_pad = jnp.zeros((8, D), f32).at[:2].set(Wxy)
    wpT = jnp.zeros((8, D), f32).at[0].set(Wp[:, 0])
    scores_col = token_scores.reshape(L * BT, 1).astype(f32)

    pe_all = _gather_pe(pos_values, gidx_col.reshape(L * BT))

    fv_ref = jax.new_ref(
        jnp.concatenate([feature_values, jnp.zeros((8, D), f32)], axis=0))
    HT = 2 * T
    q = [_gather_half(fv_ref, lax.dynamic_slice_in_dim(gidx[0], h * HT, HT))
         for h in (0, 1)]
    for l in range(L):
        for h in (0, 1):
            x = _layers[l][h](q[h], pe_all, xy_pad, scores_col, Wq, Wk, Wv,
                              Wo, W1, W2, wxy_pad, wpT)
            s_h = lax.dynamic_slice_in_dim(sidx[l], h * HT, HT)
            _sc_scatter(x, s_h, fv_ref)
            if l + 1 < L:
                g_h = lax.dynamic_slice_in_dim(gidx[l + 1], h * HT, HT)
                q[h] = _gather_half(fv_ref, g_h)
    return jax.freeze(fv_ref)[:N]


# speedup vs baseline: 1.1621x; 1.1621x over previous
import functools
import math

import jax
import jax.numpy as jnp
from jax import lax
from jax.experimental import pallas as pl
from jax.experimental.pallas import tpu as pltpu
from jax.experimental.pallas import tpu_sc as plsc

B = 4
T = 1024
L = 6
D = 256
H = 8
DH = 32
DFF = 1024
LEVEL = 5120
N = B * LEVEL
BT = B * T

NC = 2
NS = 16
NW = NC * NS
CH = 128

_SC_MESH = dict(core_axis_name="c", subcore_axis_name="s", num_cores=NC,
                num_subcores=NS)


def _prep_body(row_ref, col_ref, g_ref, s_ref):
    p = pl.program_id(0)
    off = (p % B) * LEVEL
    trow = row_ref[0]
    tcol = col_ref[...]
    eq = tcol == trow
    it = lax.broadcasted_iota(jnp.int32, (T, T), 0)
    jt = lax.broadcasted_iota(jnp.int32, (T, T), 1)
    loser = jnp.any(eq & (jt > it), axis=1, keepdims=True)
    g = tcol + off
    g_ref[...] = g
    s_ref[...] = jnp.where(loser, N, g)


_index_prep = pl.pallas_call(
    _prep_body,
    grid=(L * B,),
    in_specs=[
        pl.BlockSpec((1, 1, T), lambda p: (p, 0, 0)),
        pl.BlockSpec((T, 1), lambda p: (p, 0)),
    ],
    out_specs=[
        pl.BlockSpec((T, 1), lambda p: (p, 0)),
        pl.BlockSpec((T, 1), lambda p: (p, 0)),
    ],
    out_shape=[
        jax.ShapeDtypeStruct((L * BT, 1), jnp.int32),
        jax.ShapeDtypeStruct((L * BT, 1), jnp.int32),
    ],
)


def _make_sc_gather(M):
    rw = M // NW
    ch = min(CH, rw)
    nch = rw // ch

    @functools.partial(
        pl.kernel,
        out_type=jax.ShapeDtypeStruct((M, D), jnp.float32),
        mesh=plsc.VectorSubcoreMesh(**_SC_MESH),
        scratch_types=[
            pltpu.VMEM((ch,), jnp.int32),
            pltpu.VMEM((ch, D), jnp.float32),
            pltpu.SemaphoreType.DMA,
        ],
    )
    def gk(table, idx, out, idx_v, rows_v, sem):
        wid = lax.axis_index("s") * NC + lax.axis_index("c")
        for c in range(nch):
            base = wid * rw + c * ch
            pltpu.sync_copy(idx.at[pl.ds(base, ch)], idx_v)
            pltpu.async_copy(table.at[idx_v], rows_v, sem).wait()
            pltpu.sync_copy(rows_v, out.at[pl.ds(base, ch)])

    return gk


_gather_half = _make_sc_gather(2 * T)
_gather_pe = _make_sc_gather(L * BT)

RW = 2 * T // NW


@functools.partial(
    pl.kernel,
    out_type=(),
    mesh=plsc.VectorSubcoreMesh(**_SC_MESH),
    scratch_types=[
        pltpu.VMEM((RW,), jnp.int32),
        pltpu.VMEM((RW, D), jnp.float32),
        pltpu.SemaphoreType.DMA,
    ],
)
def _sc_scatter(x, sidx, table, idx_v, rows_v, sem):
    wid = lax.axis_index("s") * NC + lax.axis_index("c")
    base = wid * RW
    pltpu.sync_copy(sidx.at[pl.ds(base, RW)], idx_v)
    pltpu.sync_copy(x.at[pl.ds(base, RW)], rows_v)
    pltpu.async_copy(rows_v, table.at[idx_v], sem).wait()


@functools.partial(
    pl.kernel,
    out_type=jax.ShapeDtypeStruct((2 * T, D), jnp.float32),
    mesh=plsc.VectorSubcoreMesh(**_SC_MESH),
    scratch_types=[
        pltpu.VMEM((RW,), jnp.int32),
        pltpu.VMEM((RW,), jnp.int32),
        pltpu.VMEM((RW, D), jnp.float32),
        pltpu.VMEM((RW, D), jnp.float32),
        pltpu.SemaphoreType.DMA,
    ],
)
def _sc_scatter_gather(x, sidx, gidx, table, qout, sidx_v, gidx_v, srows_v,
                       grows_v, sem):
    c = lax.axis_index("c")
    s = lax.axis_index("s")
    base = c * T + s * RW
    pltpu.sync_copy(sidx.at[pl.ds(base, RW)], sidx_v)
    pltpu.sync_copy(gidx.at[pl.ds(base, RW)], gidx_v)
    pltpu.sync_copy(x.at[pl.ds(base, RW)], srows_v)
    pltpu.async_copy(srows_v, table.at[sidx_v], sem).wait()
    plsc.subcore_barrier()
    pltpu.async_copy(table.at[gidx_v], grows_v, sem).wait()
    pltpu.sync_copy(grows_v, qout.at[pl.ds(base, RW)])


def _ln(x):
    m = jnp.mean(x, axis=1, keepdims=True)
    c = x - m
    v = jnp.mean(c * c, axis=1, keepdims=True)
    return c * lax.rsqrt(v + 1e-5)


def _layer_body(q_ref, pe_ref, xy_ref, sc_ref, wq_ref, wk_ref, wv_ref,
                wo_ref, w1_ref, w2_ref, wxy_ref, wp_ref, o_ref):
    f32 = jnp.float32
    bf = jnp.bfloat16
    q = q_ref[...]
    pe = pe_ref[...] + jnp.dot(xy_ref[...], wxy_ref[...],
                               preferred_element_type=f32)
    qk = (q + pe).astype(bf)
    wq = wq_ref[0].astype(bf)
    wk = wk_ref[0].astype(bf)
    wv = wv_ref[0].astype(bf)
    wo = wo_ref[0].astype(bf)
    Q = jnp.dot(qk, wq, preferred_element_type=f32)
    K = jnp.dot(qk, wk, preferred_element_type=f32).astype(bf)
    V = jnp.dot(q.astype(bf), wv, preferred_element_type=f32).astype(bf)
    logit = jnp.sum(q * wp_ref[0:1, :], axis=1, keepdims=True)
    gate = sc_ref[...] * jax.nn.sigmoid(logit)
    scale = 1.0 / math.sqrt(DH)
    outs = []
    for h in range(H):
        sl = slice(h * DH, (h + 1) * DH)
        S = lax.dot_general((Q[:, sl] * scale).astype(bf), K[:, sl],
                            (((1,), (1,)), ((), ())),
                            preferred_element_type=f32)
        E = jnp.exp(S)
        P = (E / jnp.sum(E, axis=1, keepdims=True)).astype(bf)
        outs.append(jnp.dot(P, V[:, sl], preferred_element_type=f32))
    attn = jnp.concatenate(outs, axis=1).astype(bf)
    out = jnp.dot(attn, wo, preferred_element_type=f32)
    x = _ln(q + out * gate)
    ff = jnp.dot(jax.nn.relu(jnp.dot(x.astype(bf), w1_ref[0].astype(bf),
                                     preferred_element_type=f32)).astype(bf),
                 w2_ref[0].astype(bf), preferred_element_type=f32)
    o_ref[...] = _ln(x + ff)


def _make_layer(l, h):
    return pl.pallas_call(
        _layer_body,
        grid=(2,),
        in_specs=[
            pl.BlockSpec((T, D), lambda b: (b, 0)),
            pl.BlockSpec((T, D), lambda b: (l * B + 2 * h + b, 0)),
            pl.BlockSpec((T, 8), lambda b: (l * B + 2 * h + b, 0)),
            pl.BlockSpec((T, 1), lambda b: (l * B + 2 * h + b, 0)),
            pl.BlockSpec((1, D, D), lambda b: (l, 0, 0)),
            pl.BlockSpec((1, D, D), lambda b: (l, 0, 0)),
            pl.BlockSpec((1, D, D), lambda b: (l, 0, 0)),
            pl.BlockSpec((1, D, D), lambda b: (l, 0, 0)),
            pl.BlockSpec((1, D, DFF), lambda b: (l, 0, 0)),
            pl.BlockSpec((1, DFF, D), lambda b: (l, 0, 0)),
            pl.BlockSpec((8, D), lambda b: (0, 0)),
            pl.BlockSpec((8, D), lambda b: (0, 0)),
        ],
        out_specs=pl.BlockSpec((T, D), lambda b: (b, 0)),
        out_shape=jax.ShapeDtypeStruct((2 * T, D), jnp.float32),
    )


_layers = [[_make_layer(l, h) for h in (0, 1)] for l in range(L)]


def kernel(feature_values, pos_values, token_indices, token_scores,
           xy_positions, Wq, Wk, Wv, Wo, W1, b1, W2, b2, g1, be1, g2, be2,
           Wxy, Wp, bp):
    f32 = jnp.float32
    tok = token_indices.astype(jnp.int32)
    gidx_col, sidx_col = _index_prep(tok.reshape(L * B, 1, T),
                                     tok.reshape(L * BT, 1))
    gidx = gidx_col.reshape(L, BT)
    sidx = sidx_col.reshape(L, BT)

    xy_pad = jnp.pad(xy_positions, ((0, 0), (0, 0), (0, 0), (0, 6)))
    xy_pad = xy_pad.reshape(L * BT, 8).astype(f32)
    wxy_pad = jnp.zeros((8, D), f32).at[:2].set(Wxy)
    wpT = jnp.zeros((8, D), f32).at[0].set(Wp[:, 0])
    scores_col = token_scores.reshape(L * BT, 1).astype(f32)

    pe_all = _gather_pe(pos_values, gidx_col.reshape(L * BT))

    fv_ref = jax.new_ref(
        jnp.concatenate([feature_values, jnp.zeros((8, D), f32)], axis=0))
    HT = 2 * T
    q = [_gather_half(fv_ref, lax.dynamic_slice_in_dim(gidx[0], h * HT, HT))
         for h in (0, 1)]
    for l in range(L):
        for h in (0, 1):
            x = _layers[l][h](q[h], pe_all, xy_pad, scores_col, Wq, Wk, Wv,
                              Wo, W1, W2, wxy_pad, wpT)
            s_h = lax.dynamic_slice_in_dim(sidx[l], h * HT, HT)
            if l + 1 < L:
                g_h = lax.dynamic_slice_in_dim(gidx[l + 1], h * HT, HT)
                q[h] = _sc_scatter_gather(x, s_h, g_h, fv_ref)
            else:
                _sc_scatter(x, s_h, fv_ref)
    return jax.freeze(fv_ref)[:N]
